# Initial kernel scaffold; baseline (speedup 1.0000x reference)
import jax
import jax.numpy as jnp
from jax.experimental import pallas as pl


def kernel(x, edge_index, Ws, bs, gammas, betas):
    raise NotImplementedError("write your pallas kernel here")


# speedup vs baseline: 9.5027x; 9.5027x over previous
import functools

import jax
import jax.numpy as jnp
from jax import lax
from jax.experimental import pallas as pl
from jax.experimental.pallas import tpu as pltpu
from jax.experimental.pallas import tpu_sc as plsc

NC = 2
NS = 16
NW = NC * NS
K = 80


CHR = 80


@functools.lru_cache(maxsize=None)
def _deg_kernel(N, E):
    EPW = E // NW
    NCH = EPW // K
    NCHR = N // CHR
    MAXC = (NCHR + NS - 1) // NS
    mesh = plsc.VectorSubcoreMesh(core_axis_name="c", subcore_axis_name="s")

    @functools.partial(
        pl.kernel,
        out_type=jax.ShapeDtypeStruct((NC, N, 16), jnp.float32),
        mesh=mesh,
        scratch_types=[
            pltpu.VMEM_SHARED((N, 16), jnp.float32),
            pltpu.VMEM((K,), jnp.int32),
            pltpu.VMEM((K, 16), jnp.float32),
            pltpu.VMEM((CHR, 16), jnp.float32),
        ],
    )
    def deg(dst_hbm, out_hbm, acc, dst_v, ones_v, zbuf):
        c = lax.axis_index("c")
        s = lax.axis_index("s")
        wid = s * NC + c

        def fill(i, _):
            ones_v[i] = jnp.ones((16,), jnp.float32)
            zbuf[i] = jnp.zeros((16,), jnp.float32)
            return 0
        lax.fori_loop(0, CHR, fill, 0)

        def zchunk(k, _):
            idx = s + k * NS

            @pl.when(idx < NCHR)
            def _():
                pltpu.sync_copy(zbuf, acc.at[pl.ds(idx * CHR, CHR)])
            return 0
        lax.fori_loop(0, MAXC, zchunk, 0)
        plsc.subcore_barrier()

        base = wid * EPW

        def chunk(i, _):
            pltpu.sync_copy(dst_hbm.at[pl.ds(base + i * K, K)], dst_v)
            pltpu.sync_copy(ones_v, acc.at[dst_v], add=True)
            return 0
        lax.fori_loop(0, NCH, chunk, 0)
        plsc.subcore_barrier()

        def rchunk(k, _):
            idx = s + k * NS

            @pl.when(idx < NCHR)
            def _():
                r0 = idx * CHR
                pltpu.sync_copy(acc.at[pl.ds(r0, CHR)], zbuf)
                pltpu.sync_copy(zbuf, out_hbm.at[c, pl.ds(r0, CHR)])
            return 0
        lax.fori_loop(0, MAXC, rchunk, 0)

    return deg


@functools.lru_cache(maxsize=None)
def _agg_kernel(N, E, D):
    EPW = E // NW
    NCH = EPW // K
    NCHR = N // CHR
    MAXC = (NCHR + NS - 1) // NS
    mesh = plsc.VectorSubcoreMesh(core_axis_name="c", subcore_axis_name="s")

    @functools.partial(
        pl.kernel,
        out_type=jax.ShapeDtypeStruct((NC, N, D), jnp.float32),
        mesh=mesh,
        scratch_types=[
            pltpu.VMEM_SHARED((N, D), jnp.float32),
            pltpu.VMEM((K,), jnp.int32),
            pltpu.VMEM((K,), jnp.int32),
            pltpu.VMEM((K, D), jnp.float32),
            pltpu.VMEM((CHR, D), jnp.float32),
            pltpu.SemaphoreType.DMA,
        ],
    )
    def agg(z_hbm, src_hbm, dst_hbm, out_hbm, acc, src_v, dst_v, rows_v,
            zbuf, sem):
        c = lax.axis_index("c")
        s = lax.axis_index("s")
        wid = s * NC + c

        def fill(i, _):
            for j in range(D // 16):
                zbuf[i, pl.ds(j * 16, 16)] = jnp.zeros((16,), jnp.float32)
            return 0
        lax.fori_loop(0, CHR, fill, 0)

        def zchunk(k, _):
            idx = s + k * NS

            @pl.when(idx < NCHR)
            def _():
                pltpu.sync_copy(zbuf, acc.at[pl.ds(idx * CHR, CHR)])
            return 0
        lax.fori_loop(0, MAXC, zchunk, 0)
        plsc.subcore_barrier()

        base = wid * EPW

        def chunk(i, _):
            off = base + i * K
            pltpu.sync_copy(src_hbm.at[pl.ds(off, K)], src_v)
            pltpu.sync_copy(dst_hbm.at[pl.ds(off, K)], dst_v)
            pltpu.async_copy(z_hbm.at[src_v], rows_v, sem).wait()
            pltpu.sync_copy(rows_v, acc.at[dst_v], add=True)
            return 0
        lax.fori_loop(0, NCH, chunk, 0)
        plsc.subcore_barrier()

        def rchunk(k, _):
            idx = s + k * NS

            @pl.when(idx < NCHR)
            def _():
                r0 = idx * CHR
                pltpu.sync_copy(acc.at[pl.ds(r0, CHR)], zbuf)
                pltpu.sync_copy(zbuf, out_hbm.at[c, pl.ds(r0, CHR)])
            return 0
        lax.fori_loop(0, MAXC, rchunk, 0)

    return agg


def _tc_init(x, w0, degp):
    N, D = x.shape

    def body(x_ref, w_ref, degp_ref, dinv_ref, z_ref):
        deg = degp_ref[0, :, 0:1] + degp_ref[1, :, 0:1] + 1.0
        dinv = lax.rsqrt(deg)
        dinv_ref[...] = dinv
        z_ref[...] = jnp.dot(x_ref[...], w_ref[...],
                             preferred_element_type=jnp.float32) * dinv

    return pl.pallas_call(
        body,
        out_shape=[jax.ShapeDtypeStruct((N, 1), jnp.float32),
                   jax.ShapeDtypeStruct((N, D), jnp.float32)],
    )(x, w0, degp)


def _tc_layer(S, z, dinv, h_prev, b, g, be, w_next, with_res):
    N, D = z.shape

    def body(*refs):
        if with_res:
            (s_ref, z_ref, dinv_ref, h_ref, b_ref, g_ref, be_ref, w_ref,
             hn_ref, zn_ref) = refs
        else:
            (s_ref, z_ref, dinv_ref, b_ref, g_ref, be_ref, w_ref,
             hn_ref, zn_ref) = refs
        dinv = dinv_ref[...]
        agg = dinv * (s_ref[0] + s_ref[1] + z_ref[...]) + b_ref[...]
        mu = jnp.mean(agg, axis=0, keepdims=True)
        var = jnp.mean((agg - mu) ** 2, axis=0, keepdims=True)
        h = (agg - mu) * lax.rsqrt(var + 1e-5) * g_ref[...] + be_ref[...]
        h = jnp.maximum(h, 0.0)
        if with_res:
            h = h + h_ref[...]
        hn_ref[...] = h
        zn_ref[...] = jnp.dot(h, w_ref[...],
                              preferred_element_type=jnp.float32) * dinv

    args = [S, z, dinv] + ([h_prev] if with_res else []) + \
        [b.reshape(1, D), g.reshape(1, D), be.reshape(1, D), w_next]
    return pl.pallas_call(
        body,
        out_shape=[jax.ShapeDtypeStruct((N, D), jnp.float32),
                   jax.ShapeDtypeStruct((N, D), jnp.float32)],
    )(*args)


def _tc_final(S, z, dinv, b):
    N, D = z.shape

    def body(s_ref, z_ref, dinv_ref, b_ref, o_ref):
        o_ref[...] = dinv_ref[...] * (s_ref[0] + s_ref[1] + z_ref[...]) \
            + b_ref[...]

    return pl.pallas_call(
        body,
        out_shape=jax.ShapeDtypeStruct((N, D), jnp.float32),
    )(S, z, dinv, b.reshape(1, D))


def kernel(x, edge_index, Ws, bs, gammas, betas):
    N, D = x.shape
    E = edge_index.shape[1]
    L = Ws.shape[0]
    src = edge_index[0]
    dst = edge_index[1]

    degp = _deg_kernel(N, E)(dst)
    dinv, z = _tc_init(x, Ws[0], degp)

    agg = _agg_kernel(N, E, D)
    h = None
    for i in range(L - 1):
        S = agg(z, src, dst)
        h, z = _tc_layer(S, z, dinv, h, bs[i], gammas[i], betas[i],
                         Ws[i + 1], with_res=(i > 0))
    S = agg(z, src, dst)
    return _tc_final(S, z, dinv, bs[L - 1])
